# baseline (speedup 1.0000x reference)
import functools

import jax
import jax.numpy as jnp
from jax import lax
from jax.experimental import pallas as pl
from jax.experimental.pallas import tpu as pltpu
from jax.experimental.pallas import tpu_sc as plsc

_N = 20000
_K = 1000
_KP = 1024
_D = 4
_SCORE_THRESH = 0.05
_NMS_THRESH = 0.5

_info = plsc.get_sparse_core_info()
_NC, _NS, _L = _info.num_cores, _info.num_subcores, _info.num_lanes
_NW = _NC * _NS
_BPW = _KP // _NW


def _gather_kernel(table_hbm, idx_hbm, out_hbm, idx_v, rows_v, sem):
    wid = lax.axis_index("s") * _NC + lax.axis_index("c")
    base = wid * _BPW
    pltpu.sync_copy(idx_hbm.at[pl.ds(base, _BPW)], idx_v)
    pltpu.async_copy(table_hbm.at[idx_v], rows_v, sem).wait()
    pltpu.sync_copy(rows_v, out_hbm.at[pl.ds(base, _BPW)])


def _run_gather(table, idx):
    mesh = plsc.VectorSubcoreMesh(core_axis_name="c", subcore_axis_name="s")
    f = functools.partial(
        pl.kernel,
        mesh=mesh,
        compiler_params=pltpu.CompilerParams(use_tc_tiling_on_sc=False),
        out_type=jax.ShapeDtypeStruct((_KP, _D), jnp.float32),
        scratch_types=[
            pltpu.VMEM((_BPW,), jnp.int32),
            pltpu.VMEM((_BPW, _D), jnp.float32),
            pltpu.SemaphoreType.DMA,
        ],
    )(_gather_kernel)
    return f(table, idx)


def _nms_kernel(ts_ref, x1_ref, y1_ref, x2_ref, y2_ref,
                os_ref, ox1_ref, oy1_ref, ox2_ref, oy2_ref):
    ts = ts_ref[...]
    x1 = x1_ref[...]
    y1 = y1_ref[...]
    x2 = x2_ref[...]
    y2 = y2_ref[...]

    areas = jnp.maximum(x2 - x1, 0.0) * jnp.maximum(y2 - y1, 0.0)
    xx1 = jnp.maximum(x1[:, None], x1[None, :])
    yy1 = jnp.maximum(y1[:, None], y1[None, :])
    xx2 = jnp.minimum(x2[:, None], x2[None, :])
    yy2 = jnp.minimum(y2[:, None], y2[None, :])
    inter = jnp.maximum(xx2 - xx1, 0.0) * jnp.maximum(yy2 - yy1, 0.0)
    union = areas[:, None] + areas[None, :] - inter
    iou = inter / jnp.maximum(union, 1e-9)

    idx = jax.lax.broadcasted_iota(jnp.int32, (_KP, _KP), 0)
    jdx = jax.lax.broadcasted_iota(jnp.int32, (_KP, _KP), 1)
    m = jnp.where((iou > _NMS_THRESH) & (jdx > idx), 1.0, 0.0)

    def cond(carry):
        _, changed, it = carry
        return changed & (it < _KP + 1)

    def body(carry):
        keep, _, it = carry
        sup = jax.lax.dot_general(
            keep.reshape(1, _KP), m, (((1,), (0,)), ((), ())),
            preferred_element_type=jnp.float32).reshape(_KP)
        new_keep = jnp.where(sup > 0.0, 0.0, 1.0)
        changed = jnp.any(new_keep != keep)
        return new_keep, changed, it + 1

    keep0 = jnp.ones((_KP,), jnp.float32)
    keep, _, _ = jax.lax.while_loop(cond, body, (keep0, jnp.bool_(True),
                                                 jnp.int32(0)))

    final = (keep > 0.0) & (ts > _SCORE_THRESH)
    os_ref[...] = jnp.where(final, ts, 0.0)
    ox1_ref[...] = jnp.where(final, x1, 0.0)
    oy1_ref[...] = jnp.where(final, y1, 0.0)
    ox2_ref[...] = jnp.where(final, x2, 0.0)
    oy2_ref[...] = jnp.where(final, y2, 0.0)


def _run_nms(ts, x1, y1, x2, y2):
    return pl.pallas_call(
        _nms_kernel,
        out_shape=tuple(jax.ShapeDtypeStruct((_KP,), jnp.float32)
                        for _ in range(5)),
    )(ts, x1, y1, x2, y2)


def kernel(boxes, scores):
    valid = scores > _SCORE_THRESH
    scores_m = jnp.where(valid, scores, -1.0)
    top_scores, top_idx = jax.lax.top_k(scores_m, _K)

    pad_idx = jnp.arange(_KP - _K, dtype=jnp.int32)
    idx = jnp.concatenate([top_idx.astype(jnp.int32), pad_idx])
    tb = _run_gather(boxes, idx)

    ts = jnp.pad(top_scores, (0, _KP - _K), constant_values=-1.0)
    os_, ox1, oy1, ox2, oy2 = _run_nms(ts, tb[:, 0], tb[:, 1],
                                       tb[:, 2], tb[:, 3])
    out = jnp.stack([os_, ox1, oy1, ox2, oy2], axis=1)[:_K]
    return out


# speedup vs baseline: 1.1690x; 1.1690x over previous
import jax
import jax.numpy as jnp
from jax.experimental import pallas as pl
from jax.experimental.pallas import tpu as pltpu

_N = 20000
_K = 1000
_KP = 1024
_SCORE_THRESH = 0.05
_NMS_THRESH = 0.5


def _nms_kernel(ts_ref, x1_ref, y1_ref, x2_ref, y2_ref,
                os_ref, ox1_ref, oy1_ref, ox2_ref, oy2_ref):
    ts = ts_ref[...]
    x1 = x1_ref[...]
    y1 = y1_ref[...]
    x2 = x2_ref[...]
    y2 = y2_ref[...]

    areas = jnp.maximum(x2 - x1, 0.0) * jnp.maximum(y2 - y1, 0.0)
    xx1 = jnp.maximum(x1[:, None], x1[None, :])
    yy1 = jnp.maximum(y1[:, None], y1[None, :])
    xx2 = jnp.minimum(x2[:, None], x2[None, :])
    yy2 = jnp.minimum(y2[:, None], y2[None, :])
    inter = jnp.maximum(xx2 - xx1, 0.0) * jnp.maximum(yy2 - yy1, 0.0)
    union = areas[:, None] + areas[None, :] - inter
    iou = inter / jnp.maximum(union, 1e-9)

    idx = jax.lax.broadcasted_iota(jnp.int32, (_KP, _KP), 0)
    jdx = jax.lax.broadcasted_iota(jnp.int32, (_KP, _KP), 1)
    m = jnp.where((iou > _NMS_THRESH) & (jdx > idx), 1.0, 0.0)

    def cond(carry):
        _, changed, it = carry
        return changed & (it < _KP + 1)

    def body(carry):
        keep, _, it = carry
        sup = jax.lax.dot_general(
            keep.reshape(1, _KP), m, (((1,), (0,)), ((), ())),
            preferred_element_type=jnp.float32).reshape(_KP)
        new_keep = jnp.where(sup > 0.0, 0.0, 1.0)
        changed = jnp.any(new_keep != keep)
        return new_keep, changed, it + 1

    keep0 = jnp.ones((_KP,), jnp.float32)
    keep, _, _ = jax.lax.while_loop(cond, body, (keep0, jnp.bool_(True),
                                                 jnp.int32(0)))

    final = (keep > 0.0) & (ts > _SCORE_THRESH)
    os_ref[...] = jnp.where(final, ts, 0.0)
    ox1_ref[...] = jnp.where(final, x1, 0.0)
    oy1_ref[...] = jnp.where(final, y1, 0.0)
    ox2_ref[...] = jnp.where(final, x2, 0.0)
    oy2_ref[...] = jnp.where(final, y2, 0.0)


def _run_nms(ts, x1, y1, x2, y2):
    return pl.pallas_call(
        _nms_kernel,
        out_shape=tuple(jax.ShapeDtypeStruct((_KP,), jnp.float32)
                        for _ in range(5)),
    )(ts, x1, y1, x2, y2)


def kernel(boxes, scores):
    valid = scores > _SCORE_THRESH
    scores_m = jnp.where(valid, scores, -1.0)
    top_scores, top_idx = jax.lax.top_k(scores_m, _K)

    idx = jnp.pad(top_idx.astype(jnp.int32), (0, _KP - _K))
    x1 = jnp.take(boxes[:, 0], idx)
    y1 = jnp.take(boxes[:, 1], idx)
    x2 = jnp.take(boxes[:, 2], idx)
    y2 = jnp.take(boxes[:, 3], idx)

    ts = jnp.pad(top_scores, (0, _KP - _K), constant_values=-1.0)
    os_, ox1, oy1, ox2, oy2 = _run_nms(ts, x1, y1, x2, y2)
    out = jnp.stack([os_, ox1, oy1, ox2, oy2], axis=1)[:_K]
    return out


# speedup vs baseline: 1.7463x; 1.4938x over previous
import jax
import jax.numpy as jnp
from jax.experimental import pallas as pl
from jax.experimental.pallas import tpu as pltpu

_N = 20000
_K = 1000
_KP = 1024
_SCORE_THRESH = 0.05
_NMS_THRESH = 0.5


def _nms_kernel(ts_ref, x1_ref, y1_ref, x2_ref, y2_ref,
                os_ref, ox1_ref, oy1_ref, ox2_ref, oy2_ref):
    ts = ts_ref[...]
    x1 = x1_ref[...]
    y1 = y1_ref[...]
    x2 = x2_ref[...]
    y2 = y2_ref[...]

    areas = jnp.maximum(x2 - x1, 0.0) * jnp.maximum(y2 - y1, 0.0)
    xx1 = jnp.maximum(x1[:, None], x1[None, :])
    yy1 = jnp.maximum(y1[:, None], y1[None, :])
    xx2 = jnp.minimum(x2[:, None], x2[None, :])
    yy2 = jnp.minimum(y2[:, None], y2[None, :])
    inter = jnp.maximum(xx2 - xx1, 0.0) * jnp.maximum(yy2 - yy1, 0.0)
    union = areas[:, None] + areas[None, :] - inter
    iou = inter / jnp.maximum(union, 1e-9)

    idx = jax.lax.broadcasted_iota(jnp.int32, (_KP, _KP), 0)
    jdx = jax.lax.broadcasted_iota(jnp.int32, (_KP, _KP), 1)
    m = jnp.where((iou > _NMS_THRESH) & (jdx > idx), 1.0, 0.0)

    def cond(carry):
        _, changed, it = carry
        return changed & (it < _KP + 1)

    def body(carry):
        keep, _, it = carry
        sup = jax.lax.dot_general(
            keep.reshape(1, _KP), m, (((1,), (0,)), ((), ())),
            preferred_element_type=jnp.float32).reshape(_KP)
        new_keep = jnp.where(sup > 0.0, 0.0, 1.0)
        changed = jnp.any(new_keep != keep)
        return new_keep, changed, it + 1

    keep0 = jnp.ones((_KP,), jnp.float32)
    keep, _, _ = jax.lax.while_loop(cond, body, (keep0, jnp.bool_(True),
                                                 jnp.int32(0)))

    final = (keep > 0.0) & (ts > _SCORE_THRESH)
    os_ref[...] = jnp.where(final, ts, 0.0)
    ox1_ref[...] = jnp.where(final, x1, 0.0)
    oy1_ref[...] = jnp.where(final, y1, 0.0)
    ox2_ref[...] = jnp.where(final, x2, 0.0)
    oy2_ref[...] = jnp.where(final, y2, 0.0)


def _run_nms(ts, x1, y1, x2, y2):
    return pl.pallas_call(
        _nms_kernel,
        out_shape=tuple(jax.ShapeDtypeStruct((_KP,), jnp.float32)
                        for _ in range(5)),
    )(ts, x1, y1, x2, y2)


def kernel(boxes, scores):
    valid = scores > _SCORE_THRESH
    scores_m = jnp.where(valid, scores, -1.0)
    top_scores, top_idx = jax.lax.top_k(scores_m, _K)

    idx = jnp.pad(top_idx.astype(jnp.int32), (0, _KP - _K))
    tb = jax.lax.gather(
        boxes, idx[:, None],
        jax.lax.GatherDimensionNumbers(
            offset_dims=(1,), collapsed_slice_dims=(0,),
            start_index_map=(0,)),
        slice_sizes=(1, 4),
        unique_indices=True, indices_are_sorted=False,
        mode=jax.lax.GatherScatterMode.PROMISE_IN_BOUNDS)

    ts = jnp.pad(top_scores, (0, _KP - _K), constant_values=-1.0)
    os_, ox1, oy1, ox2, oy2 = _run_nms(ts, tb[:, 0], tb[:, 1],
                                       tb[:, 2], tb[:, 3])
    out = jnp.stack([os_, ox1, oy1, ox2, oy2], axis=1)[:_K]
    return out


# speedup vs baseline: 1.7558x; 1.0055x over previous
import jax
import jax.numpy as jnp
from jax.experimental import pallas as pl
from jax.experimental.pallas import tpu as pltpu

_N = 20000
_K = 1000
_KP = 1024
_SCORE_THRESH = 0.05
_NMS_THRESH = 0.5


def _nms_kernel(ts_ref, x1_ref, y1_ref, x2_ref, y2_ref,
                os_ref, ox1_ref, oy1_ref, ox2_ref, oy2_ref):
    ts = ts_ref[...]
    x1 = x1_ref[...]
    y1 = y1_ref[...]
    x2 = x2_ref[...]
    y2 = y2_ref[...]

    areas = jnp.maximum(x2 - x1, 0.0) * jnp.maximum(y2 - y1, 0.0)
    xx1 = jnp.maximum(x1[:, None], x1[None, :])
    yy1 = jnp.maximum(y1[:, None], y1[None, :])
    xx2 = jnp.minimum(x2[:, None], x2[None, :])
    yy2 = jnp.minimum(y2[:, None], y2[None, :])
    inter = jnp.maximum(xx2 - xx1, 0.0) * jnp.maximum(yy2 - yy1, 0.0)
    union = areas[:, None] + areas[None, :] - inter
    iou = inter / jnp.maximum(union, 1e-9)

    idx = jax.lax.broadcasted_iota(jnp.int32, (_KP, _KP), 0)
    jdx = jax.lax.broadcasted_iota(jnp.int32, (_KP, _KP), 1)
    m = jnp.where((iou > _NMS_THRESH) & (jdx > idx), 1.0, 0.0)

    def cond(carry):
        _, changed, it = carry
        return changed & (it < _KP + 1)

    def body(carry):
        keep, _, it = carry
        sup = jax.lax.dot_general(
            keep.reshape(1, _KP), m, (((1,), (0,)), ((), ())),
            preferred_element_type=jnp.float32).reshape(_KP)
        new_keep = jnp.where(sup > 0.0, 0.0, 1.0)
        changed = jnp.any(new_keep != keep)
        return new_keep, changed, it + 1

    keep0 = jnp.ones((_KP,), jnp.float32)
    keep, _, _ = jax.lax.while_loop(cond, body, (keep0, jnp.bool_(True),
                                                 jnp.int32(0)))

    final = (keep > 0.0) & (ts > _SCORE_THRESH)
    os_ref[...] = jnp.where(final, ts, 0.0)
    ox1_ref[...] = jnp.where(final, x1, 0.0)
    oy1_ref[...] = jnp.where(final, y1, 0.0)
    ox2_ref[...] = jnp.where(final, x2, 0.0)
    oy2_ref[...] = jnp.where(final, y2, 0.0)


def _run_nms(ts, x1, y1, x2, y2, interpret=False):
    out = pl.pallas_call(
        _nms_kernel,
        out_shape=tuple(jax.ShapeDtypeStruct((_KP,), jnp.float32)
                        for _ in range(5)),
        interpret=interpret,
    )(ts, x1, y1, x2, y2)
    return out


def kernel(boxes, scores):
    valid = scores > _SCORE_THRESH
    scores_m = jnp.where(valid, scores, -1.0)
    scores_p = jnp.pad(scores_m, (0, 480), constant_values=-2.0)
    top_scores, top_idx = jax.lax.top_k(scores_p, _K)
    top_boxes = jnp.take(boxes, jnp.minimum(top_idx, _N - 1), axis=0)

    ts = jnp.pad(top_scores, (0, _KP - _K), constant_values=-1.0)
    tb = jnp.pad(top_boxes, ((0, _KP - _K), (0, 0)))
    x1, y1, x2, y2 = tb[:, 0], tb[:, 1], tb[:, 2], tb[:, 3]

    os_, ox1, oy1, ox2, oy2 = _run_nms(ts, x1, y1, x2, y2)
    out = jnp.stack([os_, ox1, oy1, ox2, oy2], axis=1)[:_K]
    return out


# speedup vs baseline: 1.7607x; 1.0028x over previous
import jax
import jax.numpy as jnp
from jax.experimental import pallas as pl
from jax.experimental.pallas import tpu as pltpu

_N = 20000
_K = 1000
_KP = 1024
_SCORE_THRESH = 0.05
_NMS_THRESH = 0.5


def _nms_kernel(ts_ref, x1_ref, y1_ref, x2_ref, y2_ref,
                os_ref, ox1_ref, oy1_ref, ox2_ref, oy2_ref):
    ts = ts_ref[...]
    x1 = x1_ref[...]
    y1 = y1_ref[...]
    x2 = x2_ref[...]
    y2 = y2_ref[...]

    areas = jnp.maximum(x2 - x1, 0.0) * jnp.maximum(y2 - y1, 0.0)
    xx1 = jnp.maximum(x1[:, None], x1[None, :])
    yy1 = jnp.maximum(y1[:, None], y1[None, :])
    xx2 = jnp.minimum(x2[:, None], x2[None, :])
    yy2 = jnp.minimum(y2[:, None], y2[None, :])
    inter = jnp.maximum(xx2 - xx1, 0.0) * jnp.maximum(yy2 - yy1, 0.0)
    union = areas[:, None] + areas[None, :] - inter
    iou = inter / jnp.maximum(union, 1e-9)

    idx = jax.lax.broadcasted_iota(jnp.int32, (_KP, _KP), 0)
    jdx = jax.lax.broadcasted_iota(jnp.int32, (_KP, _KP), 1)
    m = jnp.where((iou > _NMS_THRESH) & (jdx > idx), 1.0, 0.0)

    def cond(carry):
        _, changed, it = carry
        return changed & (it < _KP + 1)

    def body(carry):
        keep, _, it = carry
        sup = jax.lax.dot_general(
            keep.reshape(1, _KP), m, (((1,), (0,)), ((), ())),
            preferred_element_type=jnp.float32).reshape(_KP)
        new_keep = jnp.where(sup > 0.0, 0.0, 1.0)
        changed = jnp.any(new_keep != keep)
        return new_keep, changed, it + 1

    keep0 = jnp.ones((_KP,), jnp.float32)
    keep, _, _ = jax.lax.while_loop(cond, body, (keep0, jnp.bool_(True),
                                                 jnp.int32(0)))

    final = (keep > 0.0) & (ts > _SCORE_THRESH)
    os_ref[...] = jnp.where(final, ts, 0.0)
    ox1_ref[...] = jnp.where(final, x1, 0.0)
    oy1_ref[...] = jnp.where(final, y1, 0.0)
    ox2_ref[...] = jnp.where(final, x2, 0.0)
    oy2_ref[...] = jnp.where(final, y2, 0.0)


def _run_nms(ts, x1, y1, x2, y2, interpret=False):
    out = pl.pallas_call(
        _nms_kernel,
        out_shape=tuple(jax.ShapeDtypeStruct((_KP,), jnp.float32)
                        for _ in range(5)),
        interpret=interpret,
    )(ts, x1, y1, x2, y2)
    return out


def kernel(boxes, scores):
    valid = scores > _SCORE_THRESH
    scores_m = jnp.where(valid, scores, -1.0)
    top_scores, top_idx = jax.lax.top_k(scores_m, _K)
    top_boxes = jnp.take(boxes, top_idx, axis=0)

    ts = jnp.pad(top_scores, (0, _KP - _K), constant_values=-1.0)
    tb = jnp.pad(top_boxes, ((0, _KP - _K), (0, 0)))
    x1, y1, x2, y2 = tb[:, 0], tb[:, 1], tb[:, 2], tb[:, 3]

    os_, ox1, oy1, ox2, oy2 = _run_nms(ts, x1, y1, x2, y2)
    out = jnp.stack([os_, ox1, oy1, ox2, oy2], axis=1)[:_K]
    return out


# speedup vs baseline: 2.0332x; 1.1548x over previous
import jax
import jax.numpy as jnp
from jax.experimental import pallas as pl
from jax.experimental.pallas import tpu as pltpu

_N = 20000
_K = 1000
_KP = 1024
_SCORE_THRESH = 0.05
_NMS_THRESH = 0.5


def _nms_kernel(ts_ref, x1_ref, y1_ref, x2_ref, y2_ref,
                os_ref, ox1_ref, oy1_ref, ox2_ref, oy2_ref):
    ts = ts_ref[...]
    x1 = x1_ref[...]
    y1 = y1_ref[...]
    x2 = x2_ref[...]
    y2 = y2_ref[...]

    areas = jnp.maximum(x2 - x1, 0.0) * jnp.maximum(y2 - y1, 0.0)
    xx1 = jnp.maximum(x1[:, None], x1[None, :])
    yy1 = jnp.maximum(y1[:, None], y1[None, :])
    xx2 = jnp.minimum(x2[:, None], x2[None, :])
    yy2 = jnp.minimum(y2[:, None], y2[None, :])
    inter = jnp.maximum(xx2 - xx1, 0.0) * jnp.maximum(yy2 - yy1, 0.0)
    union = areas[:, None] + areas[None, :] - inter
    iou = inter / jnp.maximum(union, 1e-9)

    idx = jax.lax.broadcasted_iota(jnp.int32, (_KP, _KP), 0)
    jdx = jax.lax.broadcasted_iota(jnp.int32, (_KP, _KP), 1)
    m = jnp.where((iou > _NMS_THRESH) & (jdx > idx), 1.0, 0.0)

    def cond(carry):
        _, changed, it = carry
        return changed & (it < _KP + 1)

    def body(carry):
        keep, _, it = carry
        sup = jax.lax.dot_general(
            keep.reshape(1, _KP), m, (((1,), (0,)), ((), ())),
            preferred_element_type=jnp.float32).reshape(_KP)
        new_keep = jnp.where(sup > 0.0, 0.0, 1.0)
        changed = jnp.any(new_keep != keep)
        return new_keep, changed, it + 1

    keep0 = jnp.ones((_KP,), jnp.float32)
    keep, _, _ = jax.lax.while_loop(cond, body, (keep0, jnp.bool_(True),
                                                 jnp.int32(0)))

    final = (keep > 0.0) & (ts > _SCORE_THRESH)
    os_ref[...] = jnp.where(final, ts, 0.0)
    ox1_ref[...] = jnp.where(final, x1, 0.0)
    oy1_ref[...] = jnp.where(final, y1, 0.0)
    ox2_ref[...] = jnp.where(final, x2, 0.0)
    oy2_ref[...] = jnp.where(final, y2, 0.0)


def _run_nms(ts, x1, y1, x2, y2, interpret=False):
    out = pl.pallas_call(
        _nms_kernel,
        out_shape=tuple(jax.ShapeDtypeStruct((_KP,), jnp.float32)
                        for _ in range(5)),
        interpret=interpret,
    )(ts, x1, y1, x2, y2)
    return out


def kernel(boxes, scores):
    valid = scores > _SCORE_THRESH
    scores_m = jnp.where(valid, scores, -1.0)
    top_scores, top_idx = jax.lax.approx_max_k(scores_m, _K,
                                               recall_target=1.0)
    top_boxes = jnp.take(boxes, top_idx, axis=0)

    ts = jnp.pad(top_scores, (0, _KP - _K), constant_values=-1.0)
    tb = jnp.pad(top_boxes, ((0, _KP - _K), (0, 0)))
    x1, y1, x2, y2 = tb[:, 0], tb[:, 1], tb[:, 2], tb[:, 3]

    os_, ox1, oy1, ox2, oy2 = _run_nms(ts, x1, y1, x2, y2)
    out = jnp.stack([os_, ox1, oy1, ox2, oy2], axis=1)[:_K]
    return out
